# baseline (speedup 1.0000x reference)
import functools

import jax
import jax.numpy as jnp
from jax import lax
from jax.experimental import pallas as pl
from jax.experimental.pallas import tpu as pltpu
from jax.experimental.pallas import tpu_sc as plsc

_D_MODEL = 2048
_CHUNK = 16
_NBUF = 3


def _sc_gather(table, idx, n_tokens):
    info = plsc.get_sparse_core_info()
    nc, ns = info.num_cores, info.num_subcores
    nw = nc * ns
    per_w = n_tokens // nw
    nchunks = per_w // _CHUNK

    mesh = plsc.VectorSubcoreMesh(core_axis_name="c", subcore_axis_name="s")

    @functools.partial(
        pl.kernel,
        out_type=jax.ShapeDtypeStruct((n_tokens, _D_MODEL), jnp.float32),
        mesh=mesh,
        scratch_types=[
            pltpu.VMEM((per_w,), jnp.int32),
        ]
        + [pltpu.VMEM((_CHUNK, _D_MODEL), jnp.float32)] * _NBUF
        + [pltpu.SemaphoreType.DMA] * (2 * _NBUF),
    )
    def body(table_hbm, idx_hbm, out_hbm, idx_v, *bufs_sems):
        rows = bufs_sems[:_NBUF]
        gsem = bufs_sems[_NBUF : 2 * _NBUF]
        osem = bufs_sems[2 * _NBUF :]

        wid = lax.axis_index("s") * nc + lax.axis_index("c")
        base = wid * per_w
        pltpu.sync_copy(idx_hbm.at[pl.ds(base, per_w)], idx_v)

        def gather_start(chunk, b):
            pltpu.async_copy(
                table_hbm.at[idx_v.at[pl.ds(chunk * _CHUNK, _CHUNK)]],
                rows[b],
                gsem[b],
            )

        def gather_wait(b):
            pltpu.make_async_copy(
                table_hbm.at[idx_v.at[pl.ds(0, _CHUNK)]], rows[b], gsem[b]
            ).wait()

        def out_start(chunk, b):
            pltpu.async_copy(
                rows[b], out_hbm.at[pl.ds(base + chunk * _CHUNK, _CHUNK)], osem[b]
            )

        def out_wait(b):
            pltpu.make_async_copy(
                rows[b], out_hbm.at[pl.ds(base, _CHUNK)], osem[b]
            ).wait()

        gather_start(0, 0)
        gather_start(1, 1)

        def outer(t, carry):
            for j in range(_NBUF):
                i = _NBUF * t + j
                gather_wait(j)
                bg = (j + 2) % _NBUF

                @pl.when(i + 2 < nchunks)
                def _():
                    @pl.when(i >= 1)
                    def _():
                        out_wait(bg)

                    gather_start(i + 2, bg)

                out_start(i, j)
            return carry

        lax.fori_loop(0, nchunks // _NBUF, outer, 0)
        for i in range(_NBUF * (nchunks // _NBUF), nchunks):
            gather_wait(i % _NBUF)
            out_start(i, i % _NBUF)
        for i in range(nchunks - _NBUF, nchunks):
            out_wait(i % _NBUF)

    return body(table, idx)


def kernel(input_ids, embed_table):
    b, s = input_ids.shape
    idx = input_ids.reshape(-1).astype(jnp.int32)
    flat = _sc_gather(embed_table, idx, b * s)
    return flat.reshape(b, s, _D_MODEL)
